# baseline (speedup 1.0000x reference)
import dataclasses
import functools

import jax
import jax.numpy as jnp
from jax import lax
from jax.experimental import pallas as pl
from jax.experimental.pallas import tpu as pltpu
from jax.experimental.pallas import tpu_sc as plsc

E = 320000
N = 10000
D = 128
D2 = D // 2
LANES = 16
NWORKER = 32
SPAN = E // NWORKER
CHUNK = 80
NCHUNK = SPAN // CHUNK
SUB = (80,)


def _dot_chunk(u_rows, i_rows, out_b):
    UNROLL = 16

    @pl.loop(0, CHUNK, step=LANES)
    def _(e0):
        lane = lax.iota(jnp.int32, LANES)
        rows = e0 + lane

        def dstep(k, accs):
            accs = list(accs)
            d0 = k * UNROLL
            for j in range(UNROLL):
                col = (lane + (d0 + j)) & (D2 - 1)
                pu = plsc.load_gather(u_rows, [rows, col])
                pi = plsc.load_gather(i_rows, [rows, col])
                lo_u = plsc.bitcast(pu << 16, jnp.float32)
                lo_i = plsc.bitcast(pi << 16, jnp.float32)
                hi_u = plsc.bitcast(pu, jnp.float32)
                hi_i = plsc.bitcast(pi, jnp.float32)
                accs[(2 * j) % 4] = accs[(2 * j) % 4] + lo_u * lo_i
                accs[(2 * j + 1) % 4] = accs[(2 * j + 1) % 4] + hi_u * hi_i
            return tuple(accs)

        zero = jnp.zeros((LANES,), jnp.float32)
        accs = lax.fori_loop(0, D2 // UNROLL, dstep, (zero, zero, zero, zero))
        out_b[pl.ds(e0, LANES)] = (accs[0] + accs[1]) + (accs[2] + accs[3])


def _sc_dot_kernel(
    u_hbm, i_hbm, s_hbm, d_hbm, o_hbm,
    sidx, didx, u_sh, i_sh, u0, i0, u1, i1, out0, out1,
    sem_idx, su0, si0, su1, si1, so0, so1,
):
    sub_id = lax.axis_index("subcore")
    w = lax.axis_index("core") * 16 + sub_id
    base = w * SPAN

    @pl.when(sub_id == 0)
    def _():
        pltpu.sync_copy(u_hbm, u_sh)

    @pl.when(sub_id == 1)
    def _():
        pltpu.sync_copy(i_hbm, i_sh)

    pltpu.async_copy(s_hbm.at[pl.ds(base, SPAN)], sidx, sem_idx).wait()
    pltpu.async_copy(d_hbm.at[pl.ds(base, SPAN)], didx, sem_idx).wait()

    plsc.subcore_barrier()

    def fire(t, u_buf, i_buf, sem_u, sem_i):
        off = 0
        for sub in SUB:
            pltpu.async_copy(
                u_sh.at[sidx.at[pl.ds(t * CHUNK + off, sub)]],
                u_buf.at[pl.ds(off, sub)], sem_u)
            pltpu.async_copy(
                i_sh.at[didx.at[pl.ds(t * CHUNK + off, sub)]],
                i_buf.at[pl.ds(off, sub)], sem_i)
            off += sub

    def drain(u_buf, i_buf, sem_u, sem_i):
        pltpu.make_async_copy(u_sh.at[sidx.at[pl.ds(0, CHUNK)]], u_buf, sem_u).wait()
        pltpu.make_async_copy(i_sh.at[didx.at[pl.ds(0, CHUNK)]], i_buf, sem_i).wait()

    def fire_out(t, out_b, sem_o):
        pltpu.async_copy(out_b, o_hbm.at[pl.ds(base + t * CHUNK, CHUNK)], sem_o)

    def drain_out(out_b, sem_o):
        pltpu.make_async_copy(out_b, o_hbm.at[pl.ds(base, CHUNK)], sem_o).wait()

    fire(0, u0, i0, su0, si0)

    @pl.loop(0, NCHUNK - 1, step=2)
    def _(t):
        drain(u0, i0, su0, si0)
        fire(t + 1, u1, i1, su1, si1)

        @pl.when(t > 0)
        def _():
            drain_out(out0, so0)

        _dot_chunk(u0, i0, out0)
        fire_out(t, out0, so0)

        drain(u1, i1, su1, si1)
        fire(t + 2, u0, i0, su0, si0)

        @pl.when(t > 0)
        def _():
            drain_out(out1, so1)

        _dot_chunk(u1, i1, out1)
        fire_out(t + 1, out1, so1)

    drain(u0, i0, su0, si0)
    drain_out(out0, so0)
    _dot_chunk(u0, i0, out0)
    fire_out(NCHUNK - 1, out0, so0)

    drain_out(out0, so0)
    drain_out(out1, so1)


def _pack_bf16_pairs(x):
    b = x.astype(jnp.bfloat16).reshape(N, D2, 2)
    return lax.bitcast_convert_type(b, jnp.int32)


def kernel(ufeat, ifeat, edge_index):
    src = edge_index[0].astype(jnp.int32)
    dst = edge_index[1].astype(jnp.int32)
    upk = _pack_bf16_pairs(ufeat)
    ipk = _pack_bf16_pairs(ifeat)
    mesh = plsc.VectorSubcoreMesh(core_axis_name="core", subcore_axis_name="subcore")

    cp = pltpu.CompilerParams()
    if "needs_layout_passes" in pltpu.CompilerParams.__dataclass_fields__:
        cp = dataclasses.replace(cp, needs_layout_passes=False)
    if "use_tc_tiling_on_sc" in pltpu.CompilerParams.__dataclass_fields__:
        cp = dataclasses.replace(cp, use_tc_tiling_on_sc=False)

    run = functools.partial(
        pl.kernel,
        out_type=jax.ShapeDtypeStruct((E,), jnp.float32),
        mesh=mesh,
        compiler_params=cp,
        scratch_types=[
            pltpu.VMEM((SPAN,), jnp.int32),
            pltpu.VMEM((SPAN,), jnp.int32),
            pltpu.VMEM_SHARED((N, D2), jnp.int32),
            pltpu.VMEM_SHARED((N, D2), jnp.int32),
            pltpu.VMEM((CHUNK, D2), jnp.int32),
            pltpu.VMEM((CHUNK, D2), jnp.int32),
            pltpu.VMEM((CHUNK, D2), jnp.int32),
            pltpu.VMEM((CHUNK, D2), jnp.int32),
            pltpu.VMEM((CHUNK,), jnp.float32),
            pltpu.VMEM((CHUNK,), jnp.float32),
            pltpu.SemaphoreType.DMA,
            pltpu.SemaphoreType.DMA,
            pltpu.SemaphoreType.DMA,
            pltpu.SemaphoreType.DMA,
            pltpu.SemaphoreType.DMA,
            pltpu.SemaphoreType.DMA,
            pltpu.SemaphoreType.DMA,
        ],
    )(_sc_dot_kernel)

    out = run(upk, ipk, src, dst)
    return out.reshape(E, 1)


# speedup vs baseline: 1.0213x; 1.0213x over previous
import dataclasses
import functools

import jax
import jax.numpy as jnp
from jax import lax
from jax.experimental import pallas as pl
from jax.experimental.pallas import tpu as pltpu
from jax.experimental.pallas import tpu_sc as plsc

E = 320000
N = 10000
D = 128
D2 = D // 2
LANES = 16
NWORKER = 32
SPAN = E // NWORKER
CHUNK = 400
NCHUNK = SPAN // CHUNK
SUB = (400,)


def _dot_chunk(u_rows, i_rows, out_b):
    UNROLL = 16

    @pl.loop(0, CHUNK, step=LANES)
    def _(e0):
        lane = lax.iota(jnp.int32, LANES)
        rows = e0 + lane

        def dstep(k, accs):
            accs = list(accs)
            d0 = k * UNROLL
            for j in range(UNROLL):
                col = (lane + (d0 + j)) & (D2 - 1)
                pu = plsc.load_gather(u_rows, [rows, col])
                pi = plsc.load_gather(i_rows, [rows, col])
                lo_u = plsc.bitcast(pu << 16, jnp.float32)
                lo_i = plsc.bitcast(pi << 16, jnp.float32)
                hi_u = plsc.bitcast(pu, jnp.float32)
                hi_i = plsc.bitcast(pi, jnp.float32)
                accs[(2 * j) % 4] = accs[(2 * j) % 4] + lo_u * lo_i
                accs[(2 * j + 1) % 4] = accs[(2 * j + 1) % 4] + hi_u * hi_i
            return tuple(accs)

        zero = jnp.zeros((LANES,), jnp.float32)
        accs = lax.fori_loop(0, D2 // UNROLL, dstep, (zero, zero, zero, zero))
        out_b[pl.ds(e0, LANES)] = (accs[0] + accs[1]) + (accs[2] + accs[3])


def _sc_dot_kernel(
    u_hbm, i_hbm, s_hbm, d_hbm, o_hbm,
    sidx, didx, u0, i0, u1, i1, out0, out1,
    sem_idx, su0, si0, su1, si1, so0, so1,
):
    w = lax.axis_index("core") * 16 + lax.axis_index("subcore")
    base = w * SPAN

    pltpu.async_copy(s_hbm.at[pl.ds(base, SPAN)], sidx, sem_idx).wait()
    pltpu.async_copy(d_hbm.at[pl.ds(base, SPAN)], didx, sem_idx).wait()

    def fire(t, u_buf, i_buf, sem_u, sem_i):
        off = 0
        for sub in SUB:
            pltpu.async_copy(
                u_hbm.at[sidx.at[pl.ds(t * CHUNK + off, sub)]],
                u_buf.at[pl.ds(off, sub)], sem_u)
            pltpu.async_copy(
                i_hbm.at[didx.at[pl.ds(t * CHUNK + off, sub)]],
                i_buf.at[pl.ds(off, sub)], sem_i)
            off += sub

    def drain(u_buf, i_buf, sem_u, sem_i):
        pltpu.make_async_copy(u_hbm.at[sidx.at[pl.ds(0, CHUNK)]], u_buf, sem_u).wait()
        pltpu.make_async_copy(i_hbm.at[didx.at[pl.ds(0, CHUNK)]], i_buf, sem_i).wait()

    def fire_out(t, out_b, sem_o):
        pltpu.async_copy(out_b, o_hbm.at[pl.ds(base + t * CHUNK, CHUNK)], sem_o)

    def drain_out(out_b, sem_o):
        pltpu.make_async_copy(out_b, o_hbm.at[pl.ds(base, CHUNK)], sem_o).wait()

    fire(0, u0, i0, su0, si0)

    @pl.loop(0, NCHUNK - 1, step=2)
    def _(t):
        drain(u0, i0, su0, si0)
        fire(t + 1, u1, i1, su1, si1)

        @pl.when(t > 0)
        def _():
            drain_out(out0, so0)

        _dot_chunk(u0, i0, out0)
        fire_out(t, out0, so0)

        drain(u1, i1, su1, si1)
        fire(t + 2, u0, i0, su0, si0)

        @pl.when(t > 0)
        def _():
            drain_out(out1, so1)

        _dot_chunk(u1, i1, out1)
        fire_out(t + 1, out1, so1)

    drain(u0, i0, su0, si0)
    drain_out(out0, so0)
    _dot_chunk(u0, i0, out0)
    fire_out(NCHUNK - 1, out0, so0)

    drain_out(out0, so0)
    drain_out(out1, so1)


def _pack_bf16_pairs(x):
    b = x.astype(jnp.bfloat16).reshape(N, D2, 2)
    return lax.bitcast_convert_type(b, jnp.int32)


def kernel(ufeat, ifeat, edge_index):
    src = edge_index[0].astype(jnp.int32)
    dst = edge_index[1].astype(jnp.int32)
    upk = _pack_bf16_pairs(ufeat)
    ipk = _pack_bf16_pairs(ifeat)
    mesh = plsc.VectorSubcoreMesh(core_axis_name="core", subcore_axis_name="subcore")

    cp = pltpu.CompilerParams()
    if "needs_layout_passes" in pltpu.CompilerParams.__dataclass_fields__:
        cp = dataclasses.replace(cp, needs_layout_passes=False)
    if "use_tc_tiling_on_sc" in pltpu.CompilerParams.__dataclass_fields__:
        cp = dataclasses.replace(cp, use_tc_tiling_on_sc=False)

    run = functools.partial(
        pl.kernel,
        out_type=jax.ShapeDtypeStruct((E,), jnp.float32),
        mesh=mesh,
        compiler_params=cp,
        scratch_types=[
            pltpu.VMEM((SPAN,), jnp.int32),
            pltpu.VMEM((SPAN,), jnp.int32),
            pltpu.VMEM((CHUNK, D2), jnp.int32),
            pltpu.VMEM((CHUNK, D2), jnp.int32),
            pltpu.VMEM((CHUNK, D2), jnp.int32),
            pltpu.VMEM((CHUNK, D2), jnp.int32),
            pltpu.VMEM((CHUNK,), jnp.float32),
            pltpu.VMEM((CHUNK,), jnp.float32),
            pltpu.SemaphoreType.DMA,
            pltpu.SemaphoreType.DMA,
            pltpu.SemaphoreType.DMA,
            pltpu.SemaphoreType.DMA,
            pltpu.SemaphoreType.DMA,
            pltpu.SemaphoreType.DMA,
            pltpu.SemaphoreType.DMA,
        ],
    )(_sc_dot_kernel)

    out = run(upk, ipk, src, dst)
    return out.reshape(E, 1)
